# baseline (speedup 1.0000x reference)
import functools
import math

import jax
import jax.numpy as jnp
from jax import lax
from jax.experimental import pallas as pl
from jax.experimental.pallas import tpu as pltpu
from jax.experimental.pallas import tpu_sc as plsc

D_MODEL = 64
SCALE = math.sqrt(D_MODEL)
NC, NS, L = 2, 16, 16
NW = NC * NS
BT = 128
CBLK = 8192


def _make_stage1(V, D):
    nsteps = (V + CBLK - 1) // CBLK

    def body(tt_ref, out_ref):
        out_ref[:, 0:D] = tt_ref[...].T * SCALE

    return pl.pallas_call(
        body,
        grid=(nsteps,),
        in_specs=[pl.BlockSpec((D, CBLK), lambda c: (0, c))],
        out_specs=pl.BlockSpec((CBLK, 2 * D), lambda c: (c, 0)),
        out_shape=jax.ShapeDtypeStruct((V, 2 * D), jnp.float32),
    )


def _make_stage2(S, T, V, D):
    assert S == NW * BT and D == D_MODEL
    mesh = plsc.VectorSubcoreMesh(core_axis_name="c", subcore_axis_name="s")

    @functools.partial(
        pl.kernel,
        mesh=mesh,
        out_type=jax.ShapeDtypeStruct((T, D // 8, NW, 8, BT), jnp.float32),
        scratch_types=[
            pltpu.VMEM((T, BT), jnp.int32),
            pltpu.VMEM((BT, 2 * D), jnp.float32),
            pltpu.VMEM((BT, 2 * D), jnp.float32),
            pltpu.VMEM((D // 8, 8, BT + 1), jnp.float32),
            pltpu.VMEM((D // 8, 8, BT + 1), jnp.float32),
            pltpu.VMEM((BT, D + 1), jnp.float32),
            pltpu.SemaphoreType.DMA,
            pltpu.SemaphoreType.DMA,
            pltpu.SemaphoreType.DMA,
            pltpu.SemaphoreType.DMA,
        ],
        compiler_params=pltpu.CompilerParams(
            use_tc_tiling_on_sc=True, needs_layout_passes=False
        ),
    )
    def lookup(
        xt_hbm,
        t2_hbm,
        out_hbm,
        idx_v,
        buf0,
        buf1,
        obuf0,
        obuf1,
        sbuf,
        sem0,
        sem1,
        osem0,
        osem1,
    ):
        wid = lax.axis_index("s") * NC + lax.axis_index("c")
        pltpu.sync_copy(xt_hbm.at[:, pl.ds(wid * BT, BT)], idx_v)

        bufs = (buf0, buf1)
        sems = (sem0, sem1)
        obufs = (obuf0, obuf1)
        osems = (osem0, osem1)

        iota = lax.broadcasted_iota(jnp.int32, (L,), 0)

        def start_gather(t, b):
            pltpu.async_copy(t2_hbm.at[idx_v.at[t]], bufs[b], sems[b])

        def wait_gather(t, b):
            pltpu.make_async_copy(
                t2_hbm.at[idx_v.at[t]], bufs[b], sems[b]
            ).wait()

        def start_out(t, b):
            pltpu.async_copy(
                obufs[b].at[:, :, pl.ds(0, BT)], out_hbm.at[t, :, wid], osems[b]
            )

        def wait_out(t, b):
            pltpu.make_async_copy(
                obufs[b].at[:, :, pl.ds(0, BT)], out_hbm.at[t, :, wid], osems[b]
            ).wait()

        def unit(t, b, first):
            if not first:
                wait_out(t - 2, b)
            wait_gather(t, b)
            buf = bufs[b]
            obuf = obufs[b]

            def stage(g, c):
                vs = []
                for u in range(2):
                    bl = 2 * g + u
                    for m in range(D // L):
                        vs.append((bl, m, buf[bl, pl.ds(m * L, L)]))
                for bl, m, v in vs:
                    sbuf[bl, pl.ds(m * L, L)] = v
                return c

            lax.fori_loop(0, BT // 2, stage, 0)

            def kgroup(k, c):
                blv = iota + k * L
                for dg in range(D // 8):
                    vs = [
                        plsc.load_gather(
                            sbuf, [blv, jnp.full((L,), dg * 8 + j, jnp.int32)]
                        )
                        for j in range(8)
                    ]
                    for j, v in enumerate(vs):
                        d = dg * 8 + j
                        obuf[d // 8, d % 8, pl.ds(k * L, L)] = v
                return c

            lax.fori_loop(0, BT // L, kgroup, 0)
            start_out(t, b)

        start_gather(0, 0)
        start_gather(1, 1)
        unit(0, 0, True)
        start_gather(2, 0)
        unit(1, 1, True)

        def pair(p, c):
            t0 = 2 * p

            @pl.when(t0 + 3 < T)
            def _():
                start_gather(t0 + 3, 1)

            unit(t0 + 2, 0, False)

            @pl.when(t0 + 4 < T)
            def _():
                start_gather(t0 + 4, 0)

            unit(t0 + 3, 1, False)
            return c

        lax.fori_loop(0, (T - 2) // 2, pair, 0)
        wait_out(T - 2, 0)
        wait_out(T - 1, 1)

    return lookup


def kernel(x, table):
    S, T = x.shape
    V, D = table.shape
    xt = x.T
    t2 = _make_stage1(V, D)(table.T)
    out5 = _make_stage2(S, T, V, D)(xt, t2)
    return out5.transpose(2, 4, 0, 1, 3).reshape(S, T, D)


# speedup vs baseline: 1.0576x; 1.0576x over previous
import functools
import math

import jax
import jax.numpy as jnp
from jax import lax
from jax.experimental import pallas as pl
from jax.experimental.pallas import tpu as pltpu
from jax.experimental.pallas import tpu_sc as plsc

D_MODEL = 64
SCALE = math.sqrt(D_MODEL)
NC, NS, L = 2, 16, 16
NW = NC * NS
BT = 128
CBLK = 8192


def _make_stage1(V, D):
    nsteps = (V + CBLK - 1) // CBLK

    def body(tt_ref, out_ref):
        out_ref[:, 0:D] = tt_ref[...].T * SCALE

    return pl.pallas_call(
        body,
        grid=(nsteps,),
        in_specs=[pl.BlockSpec((D, CBLK), lambda c: (0, c))],
        out_specs=pl.BlockSpec((CBLK, 2 * D), lambda c: (c, 0)),
        out_shape=jax.ShapeDtypeStruct((V, 2 * D), jnp.float32),
    )


def _make_stage2(S, T, V, D):
    assert S == NW * BT and D == D_MODEL
    mesh = plsc.VectorSubcoreMesh(core_axis_name="c", subcore_axis_name="s")

    @functools.partial(
        pl.kernel,
        mesh=mesh,
        out_type=jax.ShapeDtypeStruct((T, D // 8, NW, 8, BT), jnp.float32),
        scratch_types=[
            pltpu.VMEM((T, BT), jnp.int32),
            pltpu.VMEM((BT, 2 * D), jnp.float32),
            pltpu.VMEM((BT, 2 * D), jnp.float32),
            pltpu.VMEM((D // 8, 8, BT + 1), jnp.float32),
            pltpu.VMEM((D // 8, 8, BT + 1), jnp.float32),
            pltpu.SemaphoreType.DMA,
            pltpu.SemaphoreType.DMA,
            pltpu.SemaphoreType.DMA,
            pltpu.SemaphoreType.DMA,
        ],
        compiler_params=pltpu.CompilerParams(
            use_tc_tiling_on_sc=True, needs_layout_passes=False
        ),
    )
    def lookup(
        xt_hbm,
        t2_hbm,
        out_hbm,
        idx_v,
        buf0,
        buf1,
        obuf0,
        obuf1,
        sem0,
        sem1,
        osem0,
        osem1,
    ):
        wid = lax.axis_index("s") * NC + lax.axis_index("c")
        pltpu.sync_copy(xt_hbm.at[:, pl.ds(wid * BT, BT)], idx_v)

        bufs = (buf0, buf1)
        sems = (sem0, sem1)
        obufs = (obuf0, obuf1)
        osems = (osem0, osem1)

        iota = lax.broadcasted_iota(jnp.int32, (L,), 0)

        def start_gather(t, b):
            pltpu.async_copy(t2_hbm.at[idx_v.at[t]], bufs[b], sems[b])

        def wait_gather(t, b):
            pltpu.make_async_copy(
                t2_hbm.at[idx_v.at[t]], bufs[b], sems[b]
            ).wait()

        def start_out(t, b):
            pltpu.async_copy(
                obufs[b].at[:, :, pl.ds(0, BT)], out_hbm.at[t, :, wid], osems[b]
            )

        def wait_out(t, b):
            pltpu.make_async_copy(
                obufs[b].at[:, :, pl.ds(0, BT)], out_hbm.at[t, :, wid], osems[b]
            ).wait()

        def unit(t, b, first):
            if not first:
                wait_out(t - 2, b)
            wait_gather(t, b)
            buf = bufs[b]
            obuf = obufs[b]

            def kgroup(k, c):
                blv = iota + k * L
                for dg in range(D // 8):
                    vs = [
                        plsc.load_gather(
                            buf, [blv, jnp.full((L,), dg * 8 + j, jnp.int32)]
                        )
                        for j in range(8)
                    ]
                    for j, v in enumerate(vs):
                        d = dg * 8 + j
                        obuf[d // 8, d % 8, pl.ds(k * L, L)] = v
                return c

            lax.fori_loop(0, BT // L, kgroup, 0)
            start_out(t, b)

        start_gather(0, 0)
        start_gather(1, 1)
        unit(0, 0, True)
        start_gather(2, 0)
        unit(1, 1, True)

        def pair(p, c):
            t0 = 2 * p

            @pl.when(t0 + 3 < T)
            def _():
                start_gather(t0 + 3, 1)

            unit(t0 + 2, 0, False)

            @pl.when(t0 + 4 < T)
            def _():
                start_gather(t0 + 4, 0)

            unit(t0 + 3, 1, False)
            return c

        lax.fori_loop(0, (T - 2) // 2, pair, 0)
        wait_out(T - 2, 0)
        wait_out(T - 1, 1)

    return lookup


def kernel(x, table):
    S, T = x.shape
    V, D = table.shape
    xt = x.T
    t2 = _make_stage1(V, D)(table.T)
    out5 = _make_stage2(S, T, V, D)(xt, t2)
    return out5.transpose(2, 4, 0, 1, 3).reshape(S, T, D)
